# baseline (speedup 1.0000x reference)
import jax
import jax.numpy as jnp
from jax import lax
from jax.experimental import pallas as pl
from jax.experimental.pallas import tpu as pltpu

_ANCHOR_W = (1.08, 3.42, 6.63)
_ANCHOR_H = (1.19, 4.41, 11.38)


_NQ = 4


def _make_body(g, n_anchors, n_ch):
    gq = g // _NQ

    def chan_body(c, q, stride_ref, x_refs, o_ref):
        if c == 0:
            jvec = lax.broadcasted_iota(
                jnp.int32, (8, g), 1).astype(jnp.float32)
        for a in range(n_anchors):
            x_ref = x_refs[a]
            if c in (0, 1):
                scale = stride_ref[0]
            elif c == 2:
                scale = _ANCHOR_W[a]
            elif c == 3:
                scale = _ANCHOR_H[a]
            for ii in range(gq):
                i = q * gq + ii
                t = x_ref[ii]
                if c == 0:
                    r = (1.0 / (1.0 + jnp.exp(-t)) + jvec) * scale
                elif c == 1:
                    r = (1.0 / (1.0 + jnp.exp(-t)) + jnp.float32(i)) * scale
                elif c in (2, 3):
                    r = jnp.minimum(jnp.exp(t), 1000.0) * scale
                elif c in (4, 5):
                    r = t
                else:
                    r = 1.0 / (1.0 + jnp.exp(-t))
                o_ref[:, pl.ds((a * g + i) * g, g)] = r

    def body(stride_ref, *refs):
        o_ref = refs[-1]
        x_refs = refs[:-1]
        c_idx = pl.program_id(0)
        q_idx = pl.program_id(2)
        for c in range(n_ch):
            if c == 5:
                continue
            ccond = (c_idx == c) if c != 4 else (
                jnp.logical_or(c_idx == 4, c_idx == 5))
            for q in range(_NQ):

                @pl.when(jnp.logical_and(ccond, q_idx == q))
                def _(c=c, q=q):
                    chan_body(c, q, stride_ref, x_refs, o_ref)

    return body


def _decode(x_t, stride1, g, n_anchors, n_ch):
    nb = x_t.shape[2]
    npos = n_anchors * g * g
    grid = (n_ch, nb // 8, _NQ)
    gq = g // _NQ

    def in_map(a):
        return lambda c, bb, q, a=a: (a * n_ch + c, q, bb, 0)

    in_specs = [pl.BlockSpec(memory_space=pltpu.SMEM)]
    in_specs += [
        pl.BlockSpec((None, gq, 8, g), in_map(a)) for a in range(n_anchors)
    ]

    return pl.pallas_call(
        _make_body(g, n_anchors, n_ch),
        grid=grid,
        in_specs=in_specs,
        out_specs=pl.BlockSpec(
            (None, 8, npos),
            lambda c, bb, q: (c, bb, 0),
        ),
        out_shape=jax.ShapeDtypeStruct((n_ch, nb, npos), jnp.float32),
        compiler_params=pltpu.CompilerParams(
            dimension_semantics=("arbitrary", "arbitrary", "arbitrary"),
        ),
    )(stride1, *([x_t] * n_anchors))


def kernel(x, targets, img_size):
    del targets
    num_samples, cin, g, g2 = x.shape
    assert g == g2
    n_ch = 10
    n_anchors = cin // n_ch

    stride1 = (jnp.asarray(img_size, jnp.float32) / g).reshape(1)

    x_t = jnp.transpose(x, (1, 2, 0, 3))
    out_t = _decode(x_t, stride1, g, n_anchors, n_ch)
    return jnp.transpose(out_t, (1, 2, 0))


# speedup vs baseline: 2.7645x; 2.7645x over previous
import jax
import jax.numpy as jnp
from jax import lax
from jax.experimental import pallas as pl
from jax.experimental.pallas import tpu as pltpu

_ANCHOR_W = (1.08, 3.42, 6.63)
_ANCHOR_H = (1.19, 4.41, 11.38)


def _make_body(g, n_anchors, n_ch):
    def chan_body(c, stride_ref, x_refs, o_ref):
        if c == 0:
            jvec = lax.broadcasted_iota(
                jnp.int32, (8, g), 1).astype(jnp.float32)
        for a in range(n_anchors):
            x_ref = x_refs[a]
            if c in (0, 1):
                scale = stride_ref[0]
            elif c == 2:
                scale = _ANCHOR_W[a]
            elif c == 3:
                scale = _ANCHOR_H[a]
            for i in range(g):
                t = x_ref[i]
                if c == 0:
                    r = (1.0 / (1.0 + jnp.exp(-t)) + jvec) * scale
                elif c == 1:
                    r = (1.0 / (1.0 + jnp.exp(-t)) + jnp.float32(i)) * scale
                elif c in (2, 3):
                    r = jnp.minimum(jnp.exp(t), 1000.0) * scale
                elif c in (4, 5):
                    r = t
                else:
                    r = 1.0 / (1.0 + jnp.exp(-t))
                o_ref[:, pl.ds((a * g + i) * g, g)] = r

    def body(stride_ref, *refs):
        o_ref = refs[-1]
        x_refs = refs[:-1]
        c_idx = pl.program_id(0)
        for c in range(n_ch):
            if c == 5:
                continue
            cond = (c_idx == c) if c != 4 else (
                jnp.logical_or(c_idx == 4, c_idx == 5))

            @pl.when(cond)
            def _(c=c):
                chan_body(c, stride_ref, x_refs, o_ref)

    return body


def _decode(x_t, stride1, g, n_anchors, n_ch):
    nb = x_t.shape[2]
    npos = n_anchors * g * g
    grid = (n_ch, nb // 8)

    def in_map(a):
        return lambda c, bb, a=a: (a * n_ch + c, 0, bb, 0)

    in_specs = [pl.BlockSpec(memory_space=pltpu.SMEM)]
    in_specs += [
        pl.BlockSpec((None, g, 8, g), in_map(a)) for a in range(n_anchors)
    ]

    return pl.pallas_call(
        _make_body(g, n_anchors, n_ch),
        grid=grid,
        in_specs=in_specs,
        out_specs=pl.BlockSpec(
            (None, 8, npos),
            lambda c, bb: (c, bb, 0),
        ),
        out_shape=jax.ShapeDtypeStruct((n_ch, nb, npos), jnp.float32),
        compiler_params=pltpu.CompilerParams(
            dimension_semantics=("arbitrary", "arbitrary"),
        ),
    )(stride1, *([x_t] * n_anchors))


def kernel(x, targets, img_size):
    del targets
    num_samples, cin, g, g2 = x.shape
    assert g == g2
    n_ch = 10
    n_anchors = cin // n_ch

    stride1 = (jnp.asarray(img_size, jnp.float32) / g).reshape(1)

    x_t = jnp.transpose(x, (1, 2, 0, 3))
    out_t = _decode(x_t, stride1, g, n_anchors, n_ch)
    return jnp.transpose(out_t, (1, 2, 0))


# speedup vs baseline: 3.8028x; 1.3756x over previous
import jax
import jax.numpy as jnp
from jax import lax
from jax.experimental import pallas as pl
from jax.experimental.pallas import tpu as pltpu

_ANCHOR_W = (1.08, 3.42, 6.63)
_ANCHOR_H = (1.19, 4.41, 11.38)
_BB = 16


def _make_body(g, n_anchors, n_ch):
    def chan_body(c, stride_ref, x_refs, o_ref):
        if c == 0:
            jvec = lax.broadcasted_iota(
                jnp.int32, (_BB, g), 1).astype(jnp.float32)
        for a in range(n_anchors):
            x_ref = x_refs[a]
            if c in (0, 1):
                scale = stride_ref[0]
            elif c == 2:
                scale = _ANCHOR_W[a]
            elif c == 3:
                scale = _ANCHOR_H[a]
            for i in range(g):
                t = x_ref[i]
                if c == 0:
                    r = (1.0 / (1.0 + jnp.exp(-t)) + jvec) * scale
                elif c == 1:
                    r = (1.0 / (1.0 + jnp.exp(-t)) + jnp.float32(i)) * scale
                elif c in (2, 3):
                    r = jnp.minimum(jnp.exp(t), 1000.0) * scale
                elif c in (4, 5):
                    r = t
                else:
                    r = 1.0 / (1.0 + jnp.exp(-t))
                o_ref[:, pl.ds((a * g + i) * g, g)] = r

    def body(stride_ref, *refs):
        o_ref = refs[-1]
        x_refs = refs[:-1]
        c_idx = pl.program_id(0)
        for c in range(n_ch):
            if c == 5:
                continue
            cond = (c_idx == c) if c != 4 else (
                jnp.logical_or(c_idx == 4, c_idx == 5))

            @pl.when(cond)
            def _(c=c):
                chan_body(c, stride_ref, x_refs, o_ref)

    return body


def _decode(x_t, stride1, g, n_anchors, n_ch):
    nb = x_t.shape[2]
    npos = n_anchors * g * g
    grid = (n_ch, nb // _BB)

    def in_map(a):
        return lambda c, bb, a=a: (a * n_ch + c, 0, bb, 0)

    in_specs = [pl.BlockSpec(memory_space=pltpu.SMEM)]
    in_specs += [
        pl.BlockSpec((None, g, _BB, g), in_map(a)) for a in range(n_anchors)
    ]

    return pl.pallas_call(
        _make_body(g, n_anchors, n_ch),
        grid=grid,
        in_specs=in_specs,
        out_specs=pl.BlockSpec(
            (None, _BB, npos),
            lambda c, bb: (c, bb, 0),
        ),
        out_shape=jax.ShapeDtypeStruct((n_ch, nb, npos), jnp.float32),
        compiler_params=pltpu.CompilerParams(
            dimension_semantics=("arbitrary", "arbitrary"),
        ),
    )(stride1, *([x_t] * n_anchors))


def kernel(x, targets, img_size):
    del targets
    num_samples, cin, g, g2 = x.shape
    assert g == g2
    n_ch = 10
    n_anchors = cin // n_ch

    stride1 = (jnp.asarray(img_size, jnp.float32) / g).reshape(1)

    x_t = jnp.transpose(x, (1, 2, 0, 3))
    out_t = _decode(x_t, stride1, g, n_anchors, n_ch)
    return jnp.transpose(out_t, (1, 2, 0))


# speedup vs baseline: 4.6895x; 1.2332x over previous
import jax
import jax.numpy as jnp
from jax import lax
from jax.experimental import pallas as pl
from jax.experimental.pallas import tpu as pltpu

_ANCHOR_W = (1.08, 3.42, 6.63)
_ANCHOR_H = (1.19, 4.41, 11.38)
_BB = 32


def _make_body(g, n_anchors, n_ch):
    def chan_body(c, stride_ref, x_refs, o_ref):
        if c == 0:
            jvec = lax.broadcasted_iota(
                jnp.int32, (_BB, g), 1).astype(jnp.float32)
        for a in range(n_anchors):
            x_ref = x_refs[a]
            if c in (0, 1):
                scale = stride_ref[0]
            elif c == 2:
                scale = _ANCHOR_W[a]
            elif c == 3:
                scale = _ANCHOR_H[a]
            for i in range(g):
                t = x_ref[i]
                if c == 0:
                    r = (1.0 / (1.0 + jnp.exp(-t)) + jvec) * scale
                elif c == 1:
                    r = (1.0 / (1.0 + jnp.exp(-t)) + jnp.float32(i)) * scale
                elif c in (2, 3):
                    r = jnp.minimum(jnp.exp(t), 1000.0) * scale
                elif c in (4, 5):
                    r = t
                else:
                    r = 1.0 / (1.0 + jnp.exp(-t))
                o_ref[:, pl.ds((a * g + i) * g, g)] = r

    def body(stride_ref, *refs):
        o_ref = refs[-1]
        x_refs = refs[:-1]
        c_idx = pl.program_id(0)
        for c in range(n_ch):
            if c == 5:
                continue
            cond = (c_idx == c) if c != 4 else (
                jnp.logical_or(c_idx == 4, c_idx == 5))

            @pl.when(cond)
            def _(c=c):
                chan_body(c, stride_ref, x_refs, o_ref)

    return body


def _decode(x_t, stride1, g, n_anchors, n_ch):
    nb = x_t.shape[2]
    npos = n_anchors * g * g
    grid = (n_ch, nb // _BB)

    def in_map(a):
        return lambda c, bb, a=a: (a * n_ch + c, 0, bb, 0)

    in_specs = [pl.BlockSpec(memory_space=pltpu.SMEM)]
    in_specs += [
        pl.BlockSpec((None, g, _BB, g), in_map(a)) for a in range(n_anchors)
    ]

    return pl.pallas_call(
        _make_body(g, n_anchors, n_ch),
        grid=grid,
        in_specs=in_specs,
        out_specs=pl.BlockSpec(
            (None, _BB, npos),
            lambda c, bb: (c, bb, 0),
        ),
        out_shape=jax.ShapeDtypeStruct((n_ch, nb, npos), jnp.float32),
        compiler_params=pltpu.CompilerParams(
            dimension_semantics=("arbitrary", "arbitrary"),
        ),
    )(stride1, *([x_t] * n_anchors))


def kernel(x, targets, img_size):
    del targets
    num_samples, cin, g, g2 = x.shape
    assert g == g2
    n_ch = 10
    n_anchors = cin // n_ch

    stride1 = (jnp.asarray(img_size, jnp.float32) / g).reshape(1)

    x_t = jnp.transpose(x, (1, 2, 0, 3))
    out_t = _decode(x_t, stride1, g, n_anchors, n_ch)
    return jnp.transpose(out_t, (1, 2, 0))


# speedup vs baseline: 5.1283x; 1.0936x over previous
import jax
import jax.numpy as jnp
from jax import lax
from jax.experimental import pallas as pl
from jax.experimental.pallas import tpu as pltpu

_ANCHOR_W = (1.08, 3.42, 6.63)
_ANCHOR_H = (1.19, 4.41, 11.38)
_BB = 64


def _make_body(g, n_anchors, n_ch):
    def chan_body(c, stride_ref, x_refs, o_ref):
        if c == 0:
            jvec = lax.broadcasted_iota(
                jnp.int32, (_BB, g), 1).astype(jnp.float32)
        for a in range(n_anchors):
            x_ref = x_refs[a]
            if c in (0, 1):
                scale = stride_ref[0]
            elif c == 2:
                scale = _ANCHOR_W[a]
            elif c == 3:
                scale = _ANCHOR_H[a]
            for i in range(g):
                t = x_ref[i]
                if c == 0:
                    r = (1.0 / (1.0 + jnp.exp(-t)) + jvec) * scale
                elif c == 1:
                    r = (1.0 / (1.0 + jnp.exp(-t)) + jnp.float32(i)) * scale
                elif c in (2, 3):
                    r = jnp.minimum(jnp.exp(t), 1000.0) * scale
                elif c in (4, 5):
                    r = t
                else:
                    r = 1.0 / (1.0 + jnp.exp(-t))
                o_ref[:, pl.ds((a * g + i) * g, g)] = r

    def body(stride_ref, *refs):
        o_ref = refs[-1]
        x_refs = refs[:-1]
        c_idx = pl.program_id(0)
        for c in range(n_ch):
            if c == 5:
                continue
            cond = (c_idx == c) if c != 4 else (
                jnp.logical_or(c_idx == 4, c_idx == 5))

            @pl.when(cond)
            def _(c=c):
                chan_body(c, stride_ref, x_refs, o_ref)

    return body


def _decode(x_t, stride1, g, n_anchors, n_ch):
    nb = x_t.shape[2]
    npos = n_anchors * g * g
    grid = (n_ch, nb // _BB)

    def in_map(a):
        return lambda c, bb, a=a: (a * n_ch + c, 0, bb, 0)

    in_specs = [pl.BlockSpec(memory_space=pltpu.SMEM)]
    in_specs += [
        pl.BlockSpec((None, g, _BB, g), in_map(a)) for a in range(n_anchors)
    ]

    return pl.pallas_call(
        _make_body(g, n_anchors, n_ch),
        grid=grid,
        in_specs=in_specs,
        out_specs=pl.BlockSpec(
            (None, _BB, npos),
            lambda c, bb: (c, bb, 0),
        ),
        out_shape=jax.ShapeDtypeStruct((n_ch, nb, npos), jnp.float32),
        compiler_params=pltpu.CompilerParams(
            dimension_semantics=("arbitrary", "arbitrary"),
        ),
    )(stride1, *([x_t] * n_anchors))


def kernel(x, targets, img_size):
    del targets
    num_samples, cin, g, g2 = x.shape
    assert g == g2
    n_ch = 10
    n_anchors = cin // n_ch

    stride1 = (jnp.asarray(img_size, jnp.float32) / g).reshape(1)

    x_t = jnp.transpose(x, (1, 2, 0, 3))
    out_t = _decode(x_t, stride1, g, n_anchors, n_ch)
    return jnp.transpose(out_t, (1, 2, 0))
